# baseline (speedup 1.0000x reference)
import jax
import jax.numpy as jnp
from jax.experimental import pallas as pl
from jax.experimental.pallas import tpu as pltpu

B, T, N, F, H, K, NS, NC = 32, 12, 100, 64, 64, 2, 1, 4
M = NS * K + 1
DH = H // 2
NP = 128
NB = NP * B
C = F + H
TH = T // 2


def _dcrnn_kernel(xin_ref, sm_ref, wg0_ref, bg0_ref, wc0_ref, bc0_ref,
                  wg1_ref, bg1_ref, wc1_ref, bc1_ref,
                  a1w_ref, a1b_ref, wl_ref, awt_ref, ab_ref,
                  ewt_ref, eb_ref, f2t_ref, f2b_ref, seq_ref,
                  out_ref, xh0_ref, xh1_ref, buf_ref, land_ref, dma_sem):
    f32 = jnp.float32
    bf16 = jnp.bfloat16
    SM = sm_ref[...]

    def diffused_cat(z3):
        z0r = z3.reshape(NB, C)
        z0d = z3.reshape(NP, B * C)
        d12 = jnp.dot(SM, z0d, preferred_element_type=f32).astype(bf16)
        z1r = d12[:NP].reshape(NP, B, C).reshape(NB, C)
        z2r = d12[NP:].reshape(NP, B, C).reshape(NB, C)
        return jnp.concatenate([z0r, z1r, z2r], axis=1)

    def make_cell(xh_ref, wg_ref, bg_ref, wc_ref, bc_ref):
        def cell(xt3, h_val):
            xh_ref[:, :, :F] = xt3
            xh_ref[:, :, F:] = h_val.astype(bf16).reshape(NP, B, H)
            gates = (jnp.dot(diffused_cat(xh_ref[...]), wg_ref[...],
                             preferred_element_type=f32) + bg_ref[...])
            gates = jax.nn.sigmoid(gates)
            r = gates[:, :H]
            u = gates[:, H:]
            rh = r * h_val
            xh_ref[:, :, F:] = rh.astype(bf16).reshape(NP, B, H)
            cand = (jnp.dot(diffused_cat(xh_ref[...]), wc_ref[...],
                            preferred_element_type=f32) + bc_ref[...])
            cand = jnp.tanh(cand)
            return u * h_val + (1.0 - u) * cand

        return cell

    cell0 = make_cell(xh0_ref, wg0_ref, bg0_ref, wc0_ref, bc0_ref)
    cell1 = make_cell(xh1_ref, wg1_ref, bg1_ref, wc1_ref, bc1_ref)

    def fetch(t, slot):
        cp = pltpu.make_async_copy(xin_ref.at[t], land_ref.at[slot],
                                   dma_sem.at[slot])
        cp.start()
        return cp

    def pack(he, ho):
        return jnp.concatenate([he.astype(bf16).reshape(NP, B, H),
                                ho.astype(bf16).reshape(NP, B, H)], axis=2)

    def l0_pair(i, h):
        cp_e = fetch(2 * i, 0)
        cp_o = fetch(2 * i + 1, 1)
        cp_e.wait()
        he = cell0(land_ref[0].reshape(NP, B, F), h)
        cp_o.wait()
        ho = cell0(land_ref[1].reshape(NP, B, F), he)
        buf_ref[i] = pack(he, ho)
        return ho

    def l1_pair(i, h):
        plane = buf_ref[i]
        he = cell1(plane[:, :, :H], h)
        ho = cell1(plane[:, :, H:], he)
        buf_ref[i] = pack(he, ho)
        return ho

    h0 = l0_pair(0, jnp.zeros((NB, H), dtype=f32))

    def body(i, carry):
        h0, h1 = carry
        h0n = l0_pair(i, h0)
        h1n = l1_pair(i - 1, h1)
        return (h0n, h1n)

    h0, h1 = jax.lax.fori_loop(1, TH, body,
                               (h0, jnp.zeros((NB, H), dtype=f32)),
                               unroll=False)
    l1_pair(TH - 1, h1)

    def out_plane(t):
        plane = buf_ref[t // 2]
        o3 = plane[:, :, :H] if t % 2 == 0 else plane[:, :, H:]
        return o3.reshape(NB, H)

    planes = [out_plane(t) for t in range(T)]
    o_all = jnp.concatenate(planes, axis=0)
    oc = jax.nn.relu(jnp.dot(o_all, a1w_ref[...],
                             preferred_element_type=f32)
                     + a1b_ref[...])
    s = jnp.zeros((NB, T), dtype=f32)
    tcol = jax.lax.broadcasted_iota(jnp.int32, (NB, T), 1)
    for t in range(T):
        st = jnp.dot(oc[t * NB:(t + 1) * NB], wl_ref[...],
                     preferred_element_type=f32)
        s = s + jnp.where(tcol == t, jnp.broadcast_to(st, (NB, T)), 0.0)
    s3 = s.reshape(NP, B, T)
    lt = (jax.lax.broadcasted_iota(jnp.int32, (NP, B, T), 2)
          < seq_ref[...].reshape(1, B, 1)).astype(f32)
    s3 = s3 * (11.0 * lt - 10.0)
    mx = jnp.max(s3, axis=2, keepdims=True)
    e = jnp.exp(s3 - mx)
    aC = (e / jnp.sum(e, axis=2, keepdims=True)).reshape(NB, T)
    node = jnp.zeros((NB, H), dtype=f32)
    for t in range(T):
        node = node + aC[:, t:t + 1] * planes[t]
    node16 = node.astype(bf16)
    att = jax.nn.sigmoid(jnp.dot(node16, awt_ref[...], preferred_element_type=f32)
                         + ab_ref[...])
    emb = jnp.tanh(jnp.dot(node16, ewt_ref[...], preferred_element_type=f32)
                   + eb_ref[...])
    g = jnp.dot((att * emb).astype(bf16), f2t_ref[...],
                preferred_element_type=f32) + f2b_ref[...]
    g3 = g.reshape(NP, B, NC)
    validn = jax.lax.broadcasted_iota(jnp.int32, (NP, B, NC), 0) < N
    gmax = jnp.max(jnp.where(validn, g3, -1e30), axis=0)
    gsum = jnp.sum(jnp.where(validn, g3, 0.0), axis=0)
    out_ref[...] = gsum / float(N) + gmax


def kernel(input_seq, seq_lengths, supports, w_gate_0, b_gate_0, w_cand_0,
           b_cand_0, w_gate_1, b_gate_1, w_cand_1, b_cand_1, att1_w, att1_b,
           weight_len, attn_w, attn_b, emb_w, emb_b, fc2_w, fc2_b):
    f32 = jnp.float32
    xin = jnp.transpose(input_seq, (1, 2, 0, 3))
    xin = jnp.pad(xin, ((0, 0), (0, NP - N), (0, 0), (0, 0)))
    xin = xin.reshape(T, NP, B * F).astype(jnp.bfloat16)
    S = jnp.pad(supports[0], ((0, NP - N), (0, NP - N)))
    M2 = 2.0 * (S @ S) - jnp.eye(NP, dtype=f32)
    SM = jnp.concatenate([S, M2], axis=0).astype(jnp.bfloat16)

    def split_w(w):
        return jnp.concatenate([w[m::M] for m in range(M)],
                               axis=0).astype(jnp.bfloat16)

    wg0 = split_w(w_gate_0)
    wc0 = split_w(w_cand_0)
    wg1 = split_w(w_gate_1)
    wc1 = split_w(w_cand_1)
    seq_2d = seq_lengths.astype(jnp.int32).reshape(1, B)

    args = (xin, SM,
            wg0, b_gate_0.reshape(1, 2 * H), wc0, b_cand_0.reshape(1, H),
            wg1, b_gate_1.reshape(1, 2 * H), wc1, b_cand_1.reshape(1, H),
            att1_w.T.astype(jnp.bfloat16), att1_b.reshape(1, DH), weight_len,
            attn_w.T.astype(jnp.bfloat16), attn_b.reshape(1, H),
            emb_w.T.astype(jnp.bfloat16), emb_b.reshape(1, H),
            fc2_w.T.astype(jnp.bfloat16), fc2_b.reshape(1, NC), seq_2d)

    out = pl.pallas_call(
        _dcrnn_kernel,
        out_shape=jax.ShapeDtypeStruct((B, NC), f32),
        in_specs=[pl.BlockSpec(memory_space=pltpu.MemorySpace.HBM)]
        + [pl.BlockSpec(memory_space=pltpu.MemorySpace.VMEM)] * (len(args) - 1),
        out_specs=pl.BlockSpec(memory_space=pltpu.MemorySpace.VMEM),
        scratch_shapes=[
            pltpu.VMEM((NP, B, C), jnp.bfloat16),
            pltpu.VMEM((NP, B, C), jnp.bfloat16),
            pltpu.VMEM((TH, NP, B, 2 * H), jnp.bfloat16),
            pltpu.VMEM((2, NP, B * F), jnp.bfloat16),
            pltpu.SemaphoreType.DMA((2,)),
        ],
    )(*args)
    return out


# speedup vs baseline: 1.0904x; 1.0904x over previous
import jax
import jax.numpy as jnp
from jax.experimental import pallas as pl
from jax.experimental.pallas import tpu as pltpu

B, T, N, F, H, K, NS, NC = 32, 12, 100, 64, 64, 2, 1, 4
M = NS * K + 1
DH = H // 2
NP = 128
NB = NP * B
C = F + H
TH = T // 2


def _dcrnn_kernel(xin_ref, sm_ref, wg0_ref, bg0_ref, wc0_ref, bc0_ref,
                  wg1_ref, bg1_ref, wc1_ref, bc1_ref,
                  a1w_ref, a1b_ref, wl_ref, awt_ref, ab_ref,
                  ewt_ref, eb_ref, f2t_ref, f2b_ref, seq_ref,
                  out_ref, xh0_ref, xh1_ref, buf_ref, land_ref, dma_sem):
    f32 = jnp.float32
    bf16 = jnp.bfloat16
    SM = sm_ref[...]

    def diffused_cat(z3):
        z0r = z3.reshape(NB, C)
        z0d = z3.reshape(NP, B * C)
        d12 = jnp.dot(SM, z0d, preferred_element_type=f32).astype(bf16)
        z1r = d12[:NP].reshape(NP, B, C).reshape(NB, C)
        z2r = d12[NP:].reshape(NP, B, C).reshape(NB, C)
        return jnp.concatenate([z0r, z1r, z2r], axis=1)

    def make_cell(xh_ref, wg_ref, bg_ref, wc_ref, bc_ref):
        def cell(xt3, h_val):
            xh_ref[:, :, :F] = xt3
            xh_ref[:, :, F:] = h_val.astype(bf16).reshape(NP, B, H)
            gates = (jnp.dot(diffused_cat(xh_ref[...]), wg_ref[...],
                             preferred_element_type=f32) + bg_ref[...])
            gates = jax.nn.sigmoid(gates)
            r = gates[:, :H]
            u = gates[:, H:]
            rh = r * h_val
            xh_ref[:, :, F:] = rh.astype(bf16).reshape(NP, B, H)
            cand = (jnp.dot(diffused_cat(xh_ref[...]), wc_ref[...],
                            preferred_element_type=f32) + bc_ref[...])
            cand = jnp.tanh(cand)
            return u * h_val + (1.0 - u) * cand

        return cell

    cell0 = make_cell(xh0_ref, wg0_ref, bg0_ref, wc0_ref, bc0_ref)
    cell1 = make_cell(xh1_ref, wg1_ref, bg1_ref, wc1_ref, bc1_ref)

    def fetch(t, slot):
        cp = pltpu.make_async_copy(xin_ref.at[t], land_ref.at[slot],
                                   dma_sem.at[slot])
        cp.start()
        return cp

    def pack(he, ho):
        return jnp.concatenate([he.astype(bf16).reshape(NP, B, H),
                                ho.astype(bf16).reshape(NP, B, H)], axis=2)

    def l0_pair(i, h):
        cp_e = fetch(2 * i, 0)
        cp_o = fetch(2 * i + 1, 1)
        cp_e.wait()
        he = cell0(land_ref[0].reshape(NP, B, F), h)
        cp_o.wait()
        ho = cell0(land_ref[1].reshape(NP, B, F), he)
        buf_ref[i] = pack(he, ho)
        return ho

    def l1_pair(i, h):
        plane = buf_ref[i]
        he = cell1(plane[:, :, :H], h)
        ho = cell1(plane[:, :, H:], he)
        buf_ref[i] = pack(he, ho)
        return ho

    jax.lax.fori_loop(0, TH, lambda i, h: l0_pair(i, h),
                      jnp.zeros((NB, H), dtype=f32), unroll=False)
    jax.lax.fori_loop(0, TH, lambda i, h: l1_pair(i, h),
                      jnp.zeros((NB, H), dtype=f32), unroll=False)

    def out_plane(t):
        plane = buf_ref[t // 2]
        o3 = plane[:, :, :H] if t % 2 == 0 else plane[:, :, H:]
        return o3.reshape(NB, H)

    planes = [out_plane(t) for t in range(T)]
    o_all = jnp.concatenate(planes, axis=0)
    oc = jax.nn.relu(jnp.dot(o_all, a1w_ref[...],
                             preferred_element_type=f32)
                     + a1b_ref[...])
    s = jnp.zeros((NB, T), dtype=f32)
    tcol = jax.lax.broadcasted_iota(jnp.int32, (NB, T), 1)
    for t in range(T):
        st = jnp.dot(oc[t * NB:(t + 1) * NB], wl_ref[...],
                     preferred_element_type=f32)
        s = s + jnp.where(tcol == t, jnp.broadcast_to(st, (NB, T)), 0.0)
    s3 = s.reshape(NP, B, T)
    lt = (jax.lax.broadcasted_iota(jnp.int32, (NP, B, T), 2)
          < seq_ref[...].reshape(1, B, 1)).astype(f32)
    s3 = s3 * (11.0 * lt - 10.0)
    mx = jnp.max(s3, axis=2, keepdims=True)
    e = jnp.exp(s3 - mx)
    aC = (e / jnp.sum(e, axis=2, keepdims=True)).reshape(NB, T)
    node = jnp.zeros((NB, H), dtype=f32)
    for t in range(T):
        node = node + aC[:, t:t + 1] * planes[t]
    node16 = node.astype(bf16)
    att = jax.nn.sigmoid(jnp.dot(node16, awt_ref[...], preferred_element_type=f32)
                         + ab_ref[...])
    emb = jnp.tanh(jnp.dot(node16, ewt_ref[...], preferred_element_type=f32)
                   + eb_ref[...])
    g = jnp.dot((att * emb).astype(bf16), f2t_ref[...],
                preferred_element_type=f32) + f2b_ref[...]
    g3 = g.reshape(NP, B, NC)
    validn = jax.lax.broadcasted_iota(jnp.int32, (NP, B, NC), 0) < N
    gmax = jnp.max(jnp.where(validn, g3, -1e30), axis=0)
    gsum = jnp.sum(jnp.where(validn, g3, 0.0), axis=0)
    out_ref[...] = gsum / float(N) + gmax


def kernel(input_seq, seq_lengths, supports, w_gate_0, b_gate_0, w_cand_0,
           b_cand_0, w_gate_1, b_gate_1, w_cand_1, b_cand_1, att1_w, att1_b,
           weight_len, attn_w, attn_b, emb_w, emb_b, fc2_w, fc2_b):
    f32 = jnp.float32
    xin = jnp.transpose(input_seq, (1, 2, 0, 3))
    xin = jnp.pad(xin, ((0, 0), (0, NP - N), (0, 0), (0, 0)))
    xin = xin.reshape(T, NP, B * F).astype(jnp.bfloat16)
    S = jnp.pad(supports[0], ((0, NP - N), (0, NP - N)))
    M2 = 2.0 * (S @ S) - jnp.eye(NP, dtype=f32)
    SM = jnp.concatenate([S, M2], axis=0).astype(jnp.bfloat16)

    def split_w(w):
        return jnp.concatenate([w[m::M] for m in range(M)],
                               axis=0).astype(jnp.bfloat16)

    wg0 = split_w(w_gate_0)
    wc0 = split_w(w_cand_0)
    wg1 = split_w(w_gate_1)
    wc1 = split_w(w_cand_1)
    seq_2d = seq_lengths.astype(jnp.int32).reshape(1, B)

    args = (xin, SM,
            wg0, b_gate_0.reshape(1, 2 * H), wc0, b_cand_0.reshape(1, H),
            wg1, b_gate_1.reshape(1, 2 * H), wc1, b_cand_1.reshape(1, H),
            att1_w.T.astype(jnp.bfloat16), att1_b.reshape(1, DH), weight_len,
            attn_w.T.astype(jnp.bfloat16), attn_b.reshape(1, H),
            emb_w.T.astype(jnp.bfloat16), emb_b.reshape(1, H),
            fc2_w.T.astype(jnp.bfloat16), fc2_b.reshape(1, NC), seq_2d)

    out = pl.pallas_call(
        _dcrnn_kernel,
        out_shape=jax.ShapeDtypeStruct((B, NC), f32),
        in_specs=[pl.BlockSpec(memory_space=pltpu.MemorySpace.HBM)]
        + [pl.BlockSpec(memory_space=pltpu.MemorySpace.VMEM)] * (len(args) - 1),
        out_specs=pl.BlockSpec(memory_space=pltpu.MemorySpace.VMEM),
        scratch_shapes=[
            pltpu.VMEM((NP, B, C), jnp.bfloat16),
            pltpu.VMEM((NP, B, C), jnp.bfloat16),
            pltpu.VMEM((TH, NP, B, 2 * H), jnp.bfloat16),
            pltpu.VMEM((2, NP, B * F), jnp.bfloat16),
            pltpu.SemaphoreType.DMA((2,)),
        ],
    )(*args)
    return out


# speedup vs baseline: 1.1315x; 1.0377x over previous
import jax
import jax.numpy as jnp
from jax.experimental import pallas as pl
from jax.experimental.pallas import tpu as pltpu

B, T, N, F, H, K, NS, NC = 32, 12, 100, 64, 64, 2, 1, 4
M = NS * K + 1
DH = H // 2
NP = 128
NB = NP * B
C = F + H
TH = T // 2


def _dcrnn_kernel(xin_ref, sm_ref, wg0_ref, bg0_ref, wc0_ref, bc0_ref,
                  wg1_ref, bg1_ref, wc1_ref, bc1_ref,
                  a1w_ref, a1b_ref, wl_ref, awt_ref, ab_ref,
                  ewt_ref, eb_ref, f2t_ref, f2b_ref, seq_ref,
                  out_ref, buf_ref, land_ref, dma_sem):
    f32 = jnp.float32
    bf16 = jnp.bfloat16
    SM = sm_ref[...]

    def diffused_cat(z3):
        z0r = z3.reshape(NB, C)
        z0d = z3.reshape(NP, B * C)
        d12 = jnp.dot(SM, z0d, preferred_element_type=f32).astype(bf16)
        z1r = d12[:NP].reshape(NP, B, C).reshape(NB, C)
        z2r = d12[NP:].reshape(NP, B, C).reshape(NB, C)
        return jnp.concatenate([z0r, z1r, z2r], axis=1)

    def make_cell(wg_ref, bg_ref, wc_ref, bc_ref):
        def cell(xt3, h_val):
            xh3 = jnp.concatenate(
                [xt3, h_val.astype(bf16).reshape(NP, B, H)], axis=2)
            gates = (jnp.dot(diffused_cat(xh3), wg_ref[...],
                             preferred_element_type=f32) + bg_ref[...])
            gates = jax.nn.sigmoid(gates)
            r = gates[:, :H]
            u = gates[:, H:]
            rh = r * h_val
            xc3 = jnp.concatenate(
                [xt3, rh.astype(bf16).reshape(NP, B, H)], axis=2)
            cand = (jnp.dot(diffused_cat(xc3), wc_ref[...],
                            preferred_element_type=f32) + bc_ref[...])
            cand = jnp.tanh(cand)
            return u * h_val + (1.0 - u) * cand

        return cell

    cell0 = make_cell(wg0_ref, bg0_ref, wc0_ref, bc0_ref)
    cell1 = make_cell(wg1_ref, bg1_ref, wc1_ref, bc1_ref)

    def fetch(t, slot):
        cp = pltpu.make_async_copy(xin_ref.at[t], land_ref.at[slot],
                                   dma_sem.at[slot])
        cp.start()
        return cp

    def pack(he, ho):
        return jnp.concatenate([he.astype(bf16).reshape(NP, B, H),
                                ho.astype(bf16).reshape(NP, B, H)], axis=2)

    def l0_pair(i, h):
        cp_e = fetch(2 * i, 0)
        cp_o = fetch(2 * i + 1, 1)
        cp_e.wait()
        he = cell0(land_ref[0].reshape(NP, B, F), h)
        cp_o.wait()
        ho = cell0(land_ref[1].reshape(NP, B, F), he)
        buf_ref[i] = pack(he, ho)
        return ho

    def l1_pair(i, h):
        plane = buf_ref[i]
        he = cell1(plane[:, :, :H], h)
        ho = cell1(plane[:, :, H:], he)
        buf_ref[i] = pack(he, ho)
        return ho

    jax.lax.fori_loop(0, TH, lambda i, h: l0_pair(i, h),
                      jnp.zeros((NB, H), dtype=f32), unroll=False)
    jax.lax.fori_loop(0, TH, lambda i, h: l1_pair(i, h),
                      jnp.zeros((NB, H), dtype=f32), unroll=False)

    def out_plane(t):
        plane = buf_ref[t // 2]
        o3 = plane[:, :, :H] if t % 2 == 0 else plane[:, :, H:]
        return o3.reshape(NB, H)

    planes = [out_plane(t) for t in range(T)]
    o_all = jnp.concatenate(planes, axis=0)
    oc = jax.nn.relu(jnp.dot(o_all, a1w_ref[...],
                             preferred_element_type=f32)
                     + a1b_ref[...])
    s = jnp.zeros((NB, T), dtype=f32)
    tcol = jax.lax.broadcasted_iota(jnp.int32, (NB, T), 1)
    for t in range(T):
        st = jnp.dot(oc[t * NB:(t + 1) * NB], wl_ref[...],
                     preferred_element_type=f32)
        s = s + jnp.where(tcol == t, jnp.broadcast_to(st, (NB, T)), 0.0)
    s3 = s.reshape(NP, B, T)
    lt = (jax.lax.broadcasted_iota(jnp.int32, (NP, B, T), 2)
          < seq_ref[...].reshape(1, B, 1)).astype(f32)
    s3 = s3 * (11.0 * lt - 10.0)
    mx = jnp.max(s3, axis=2, keepdims=True)
    e = jnp.exp(s3 - mx)
    aC = (e / jnp.sum(e, axis=2, keepdims=True)).reshape(NB, T)
    node = jnp.zeros((NB, H), dtype=f32)
    for t in range(T):
        node = node + aC[:, t:t + 1] * planes[t]
    node16 = node.astype(bf16)
    att = jax.nn.sigmoid(jnp.dot(node16, awt_ref[...], preferred_element_type=f32)
                         + ab_ref[...])
    emb = jnp.tanh(jnp.dot(node16, ewt_ref[...], preferred_element_type=f32)
                   + eb_ref[...])
    g = jnp.dot((att * emb).astype(bf16), f2t_ref[...],
                preferred_element_type=f32) + f2b_ref[...]
    g3 = g.reshape(NP, B, NC)
    validn = jax.lax.broadcasted_iota(jnp.int32, (NP, B, NC), 0) < N
    gmax = jnp.max(jnp.where(validn, g3, -1e30), axis=0)
    gsum = jnp.sum(jnp.where(validn, g3, 0.0), axis=0)
    out_ref[...] = gsum / float(N) + gmax


def kernel(input_seq, seq_lengths, supports, w_gate_0, b_gate_0, w_cand_0,
           b_cand_0, w_gate_1, b_gate_1, w_cand_1, b_cand_1, att1_w, att1_b,
           weight_len, attn_w, attn_b, emb_w, emb_b, fc2_w, fc2_b):
    f32 = jnp.float32
    xin = jnp.transpose(input_seq, (1, 2, 0, 3))
    xin = jnp.pad(xin, ((0, 0), (0, NP - N), (0, 0), (0, 0)))
    xin = xin.reshape(T, NP, B * F).astype(jnp.bfloat16)
    S = jnp.pad(supports[0], ((0, NP - N), (0, NP - N)))
    M2 = 2.0 * (S @ S) - jnp.eye(NP, dtype=f32)
    SM = jnp.concatenate([S, M2], axis=0).astype(jnp.bfloat16)

    def split_w(w):
        return jnp.concatenate([w[m::M] for m in range(M)],
                               axis=0).astype(jnp.bfloat16)

    wg0 = split_w(w_gate_0)
    wc0 = split_w(w_cand_0)
    wg1 = split_w(w_gate_1)
    wc1 = split_w(w_cand_1)
    seq_2d = seq_lengths.astype(jnp.int32).reshape(1, B)

    args = (xin, SM,
            wg0, b_gate_0.reshape(1, 2 * H), wc0, b_cand_0.reshape(1, H),
            wg1, b_gate_1.reshape(1, 2 * H), wc1, b_cand_1.reshape(1, H),
            att1_w.T.astype(jnp.bfloat16), att1_b.reshape(1, DH), weight_len,
            attn_w.T.astype(jnp.bfloat16), attn_b.reshape(1, H),
            emb_w.T.astype(jnp.bfloat16), emb_b.reshape(1, H),
            fc2_w.T.astype(jnp.bfloat16), fc2_b.reshape(1, NC), seq_2d)

    out = pl.pallas_call(
        _dcrnn_kernel,
        out_shape=jax.ShapeDtypeStruct((B, NC), f32),
        in_specs=[pl.BlockSpec(memory_space=pltpu.MemorySpace.HBM)]
        + [pl.BlockSpec(memory_space=pltpu.MemorySpace.VMEM)] * (len(args) - 1),
        out_specs=pl.BlockSpec(memory_space=pltpu.MemorySpace.VMEM),
        scratch_shapes=[
            pltpu.VMEM((TH, NP, B, 2 * H), jnp.bfloat16),
            pltpu.VMEM((2, NP, B * F), jnp.bfloat16),
            pltpu.SemaphoreType.DMA((2,)),
        ],
    )(*args)
    return out


# speedup vs baseline: 1.1469x; 1.0136x over previous
import jax
import jax.numpy as jnp
from jax.experimental import pallas as pl
from jax.experimental.pallas import tpu as pltpu

B, T, N, F, H, K, NS, NC = 32, 12, 100, 64, 64, 2, 1, 4
M = NS * K + 1
DH = H // 2
NP = 128
NB = NP * B
C = F + H
TH = T // 2


def _dcrnn_kernel(xin_ref, sm_ref, wg0_ref, bg0_ref, wc0_ref, bc0_ref,
                  wg1_ref, bg1_ref, wc1_ref, bc1_ref,
                  a1w_ref, a1b_ref, wl_ref, awt_ref, ab_ref,
                  ewt_ref, eb_ref, f2t_ref, f2b_ref, seq_ref,
                  out_ref, buf_ref, land_ref, dma_sem):
    f32 = jnp.float32
    bf16 = jnp.bfloat16
    SM = sm_ref[...]

    def diffused_cat(z3):
        z0r = z3.reshape(NB, C)
        z0d = z3.reshape(NP, B * C)
        d12 = jnp.dot(SM, z0d, preferred_element_type=f32).astype(bf16)
        z1r = d12[:NP].reshape(NP, B, C).reshape(NB, C)
        z2r = d12[NP:].reshape(NP, B, C).reshape(NB, C)
        return jnp.concatenate([z0r, z1r, z2r], axis=1)

    def make_cell(wg_ref, bg_ref, wc_ref, bc_ref):
        def cell(xt3, h_val):
            xh3 = jnp.concatenate(
                [xt3, h_val.astype(bf16).reshape(NP, B, H)], axis=2)
            gates = (jnp.dot(diffused_cat(xh3), wg_ref[...],
                             preferred_element_type=f32) + bg_ref[...])
            gates = jax.nn.sigmoid(gates)
            r = gates[:, :H]
            u = gates[:, H:]
            rh = r * h_val
            xc3 = jnp.concatenate(
                [xt3, rh.astype(bf16).reshape(NP, B, H)], axis=2)
            cand = (jnp.dot(diffused_cat(xc3), wc_ref[...],
                            preferred_element_type=f32) + bc_ref[...])
            cand = jnp.tanh(cand)
            return u * h_val + (1.0 - u) * cand

        return cell

    cell0 = make_cell(wg0_ref, bg0_ref, wc0_ref, bc0_ref)
    cell1 = make_cell(wg1_ref, bg1_ref, wc1_ref, bc1_ref)

    def dma(t, slot):
        return pltpu.make_async_copy(xin_ref.at[t], land_ref.at[slot],
                                     dma_sem.at[slot])

    def l0_step(t, h):
        @pl.when(t + 1 < T)
        def _():
            dma(t + 1, (t + 1) % 2).start()
        dma(t, t % 2).wait()
        hn = cell0(land_ref[t % 2].reshape(NP, B, F), h)
        buf_ref[t] = hn.astype(bf16).reshape(NP, B, H)
        return hn

    def l1_step(t, h):
        hn = cell1(buf_ref[t], h)
        buf_ref[t] = hn.astype(bf16).reshape(NP, B, H)
        return hn

    dma(0, 0).start()
    jax.lax.fori_loop(0, T, l0_step, jnp.zeros((NB, H), dtype=f32),
                      unroll=False)
    jax.lax.fori_loop(0, T, l1_step, jnp.zeros((NB, H), dtype=f32),
                      unroll=False)

    def out_plane(t):
        return buf_ref[t].reshape(NB, H)

    planes = [out_plane(t) for t in range(T)]
    o_all = jnp.concatenate(planes, axis=0)
    oc = jax.nn.relu(jnp.dot(o_all, a1w_ref[...],
                             preferred_element_type=f32)
                     + a1b_ref[...])
    s = jnp.zeros((NB, T), dtype=f32)
    tcol = jax.lax.broadcasted_iota(jnp.int32, (NB, T), 1)
    for t in range(T):
        st = jnp.dot(oc[t * NB:(t + 1) * NB], wl_ref[...],
                     preferred_element_type=f32)
        s = s + jnp.where(tcol == t, jnp.broadcast_to(st, (NB, T)), 0.0)
    s3 = s.reshape(NP, B, T)
    lt = (jax.lax.broadcasted_iota(jnp.int32, (NP, B, T), 2)
          < seq_ref[...].reshape(1, B, 1)).astype(f32)
    s3 = s3 * (11.0 * lt - 10.0)
    mx = jnp.max(s3, axis=2, keepdims=True)
    e = jnp.exp(s3 - mx)
    aC = (e / jnp.sum(e, axis=2, keepdims=True)).reshape(NB, T)
    node = jnp.zeros((NB, H), dtype=f32)
    for t in range(T):
        node = node + aC[:, t:t + 1] * planes[t]
    node16 = node.astype(bf16)
    att = jax.nn.sigmoid(jnp.dot(node16, awt_ref[...], preferred_element_type=f32)
                         + ab_ref[...])
    emb = jnp.tanh(jnp.dot(node16, ewt_ref[...], preferred_element_type=f32)
                   + eb_ref[...])
    g = jnp.dot((att * emb).astype(bf16), f2t_ref[...],
                preferred_element_type=f32) + f2b_ref[...]
    g3 = g.reshape(NP, B, NC)
    validn = jax.lax.broadcasted_iota(jnp.int32, (NP, B, NC), 0) < N
    gmax = jnp.max(jnp.where(validn, g3, -1e30), axis=0)
    gsum = jnp.sum(jnp.where(validn, g3, 0.0), axis=0)
    out_ref[...] = gsum / float(N) + gmax


def kernel(input_seq, seq_lengths, supports, w_gate_0, b_gate_0, w_cand_0,
           b_cand_0, w_gate_1, b_gate_1, w_cand_1, b_cand_1, att1_w, att1_b,
           weight_len, attn_w, attn_b, emb_w, emb_b, fc2_w, fc2_b):
    f32 = jnp.float32
    xin = jnp.transpose(input_seq, (1, 2, 0, 3))
    xin = jnp.pad(xin, ((0, 0), (0, NP - N), (0, 0), (0, 0)))
    xin = xin.reshape(T, NP, B * F).astype(jnp.bfloat16)
    S = jnp.pad(supports[0], ((0, NP - N), (0, NP - N)))
    M2 = 2.0 * (S @ S) - jnp.eye(NP, dtype=f32)
    SM = jnp.concatenate([S, M2], axis=0).astype(jnp.bfloat16)

    def split_w(w):
        return jnp.concatenate([w[m::M] for m in range(M)],
                               axis=0).astype(jnp.bfloat16)

    wg0 = split_w(w_gate_0)
    wc0 = split_w(w_cand_0)
    wg1 = split_w(w_gate_1)
    wc1 = split_w(w_cand_1)
    seq_2d = seq_lengths.astype(jnp.int32).reshape(1, B)

    args = (xin, SM,
            wg0, b_gate_0.reshape(1, 2 * H), wc0, b_cand_0.reshape(1, H),
            wg1, b_gate_1.reshape(1, 2 * H), wc1, b_cand_1.reshape(1, H),
            att1_w.T.astype(jnp.bfloat16), att1_b.reshape(1, DH), weight_len,
            attn_w.T.astype(jnp.bfloat16), attn_b.reshape(1, H),
            emb_w.T.astype(jnp.bfloat16), emb_b.reshape(1, H),
            fc2_w.T.astype(jnp.bfloat16), fc2_b.reshape(1, NC), seq_2d)

    out = pl.pallas_call(
        _dcrnn_kernel,
        out_shape=jax.ShapeDtypeStruct((B, NC), f32),
        in_specs=[pl.BlockSpec(memory_space=pltpu.MemorySpace.HBM)]
        + [pl.BlockSpec(memory_space=pltpu.MemorySpace.VMEM)] * (len(args) - 1),
        out_specs=pl.BlockSpec(memory_space=pltpu.MemorySpace.VMEM),
        scratch_shapes=[
            pltpu.VMEM((T, NP, B, H), jnp.bfloat16),
            pltpu.VMEM((2, NP, B * F), jnp.bfloat16),
            pltpu.SemaphoreType.DMA((2,)),
        ],
    )(*args)
    return out
